# baseline (speedup 1.0000x reference)
import jax
import jax.numpy as jnp
from jax.experimental import pallas as pl
from jax.experimental.pallas import tpu as pltpu

N, C, H, W = 16, 512, 32, 32
BN = 2


def _fused_body(mask_ref, w1_ref, b1_ref, w2_ref, b2_ref, wfc_ref, bfc_ref,
                lst_ref, gui_ref, m_ref, out_lst_ref, out_gui_ref, sel_ref):
    n = pl.program_id(0)

    @pl.when(n == 0)
    def _mlp():
        h = jax.nn.relu(
            jnp.dot(mask_ref[...], w1_ref[...],
                    preferred_element_type=jnp.float32) + b1_ref[...])
        m = jax.nn.sigmoid(
            jnp.dot(h, w2_ref[...],
                    preferred_element_type=jnp.float32) + b2_ref[...])
        s = jax.nn.sigmoid(
            jnp.dot(m, wfc_ref[...],
                    preferred_element_type=jnp.float32) + bfc_ref[...])
        m_ref[...] = m
        sel_ref[...] = s

    rows = [sel_ref[n * BN + j, :][None, :] for j in range(BN)]
    cond = (jnp.concatenate(rows, axis=0) > 0.5)[:, None, None, :]
    l = lst_ref[...]
    g = gui_ref[...]
    out_lst_ref[...] = jnp.where(cond, g, l)
    out_gui_ref[...] = jnp.where(cond, l, g)


def kernel(lst, gui, mask, W1, b1, W2, b2, Wfc, bfc):
    lst_t = lst.transpose(0, 2, 3, 1)
    gui_t = gui.transpose(0, 2, 3, 1)

    m, out_lst_t, out_gui_t = pl.pallas_call(
        _fused_body,
        grid=(N // BN,),
        in_specs=[
            pl.BlockSpec((N, 1024), lambda n: (0, 0)),
            pl.BlockSpec((1024, C), lambda n: (0, 0)),
            pl.BlockSpec((1, C), lambda n: (0, 0)),
            pl.BlockSpec((C, C), lambda n: (0, 0)),
            pl.BlockSpec((1, C), lambda n: (0, 0)),
            pl.BlockSpec((C, C), lambda n: (0, 0)),
            pl.BlockSpec((1, C), lambda n: (0, 0)),
            pl.BlockSpec((BN, H, W, C), lambda n: (n, 0, 0, 0)),
            pl.BlockSpec((BN, H, W, C), lambda n: (n, 0, 0, 0)),
        ],
        out_specs=[
            pl.BlockSpec((N, C), lambda n: (0, 0)),
            pl.BlockSpec((BN, H, W, C), lambda n: (n, 0, 0, 0)),
            pl.BlockSpec((BN, H, W, C), lambda n: (n, 0, 0, 0)),
        ],
        out_shape=(
            jax.ShapeDtypeStruct((N, C), jnp.float32),
            jax.ShapeDtypeStruct((N, H, W, C), jnp.float32),
            jax.ShapeDtypeStruct((N, H, W, C), jnp.float32),
        ),
        scratch_shapes=[pltpu.VMEM((N, C), jnp.float32)],
        compiler_params=pltpu.CompilerParams(
            dimension_semantics=("arbitrary",)),
    )(mask, W1, b1.reshape(1, C), W2, b2.reshape(1, C),
      Wfc, bfc.reshape(1, C), lst_t, gui_t)

    return (out_lst_t.transpose(0, 3, 1, 2),
            out_gui_t.transpose(0, 3, 1, 2), m)


# speedup vs baseline: 1.0710x; 1.0710x over previous
import jax
import jax.numpy as jnp
from jax.experimental import pallas as pl
from jax.experimental.pallas import tpu as pltpu

N, C, H, W = 16, 512, 32, 32
BN = 3


def _fused_body(mask_ref, w1_ref, b1_ref, w2_ref, b2_ref, wfc_ref, bfc_ref,
                lst_ref, gui_ref, m_ref, out_lst_ref, out_gui_ref, sel_ref):
    n = pl.program_id(0)

    @pl.when(n == 0)
    def _mlp():
        h = jax.nn.relu(
            jnp.dot(mask_ref[...], w1_ref[...],
                    preferred_element_type=jnp.float32) + b1_ref[...])
        m = jax.nn.sigmoid(
            jnp.dot(h, w2_ref[...],
                    preferred_element_type=jnp.float32) + b2_ref[...])
        s = jax.nn.sigmoid(
            jnp.dot(m, wfc_ref[...],
                    preferred_element_type=jnp.float32) + bfc_ref[...])
        m_ref[...] = m
        sel_ref[...] = s

    rows = [sel_ref[jnp.minimum(n * BN + j, N - 1), :][None, :] for j in range(BN)]
    cond = (jnp.concatenate(rows, axis=0) > 0.5)[:, None, None, :]
    l = lst_ref[...]
    g = gui_ref[...]
    out_lst_ref[...] = jnp.where(cond, g, l)
    out_gui_ref[...] = jnp.where(cond, l, g)


def kernel(lst, gui, mask, W1, b1, W2, b2, Wfc, bfc):
    lst_t = lst.transpose(0, 2, 3, 1)
    gui_t = gui.transpose(0, 2, 3, 1)

    m, out_lst_t, out_gui_t = pl.pallas_call(
        _fused_body,
        grid=(-(-N // BN),),
        in_specs=[
            pl.BlockSpec((N, 1024), lambda n: (0, 0)),
            pl.BlockSpec((1024, C), lambda n: (0, 0)),
            pl.BlockSpec((1, C), lambda n: (0, 0)),
            pl.BlockSpec((C, C), lambda n: (0, 0)),
            pl.BlockSpec((1, C), lambda n: (0, 0)),
            pl.BlockSpec((C, C), lambda n: (0, 0)),
            pl.BlockSpec((1, C), lambda n: (0, 0)),
            pl.BlockSpec((BN, H, W, C), lambda n: (n, 0, 0, 0)),
            pl.BlockSpec((BN, H, W, C), lambda n: (n, 0, 0, 0)),
        ],
        out_specs=[
            pl.BlockSpec((N, C), lambda n: (0, 0)),
            pl.BlockSpec((BN, H, W, C), lambda n: (n, 0, 0, 0)),
            pl.BlockSpec((BN, H, W, C), lambda n: (n, 0, 0, 0)),
        ],
        out_shape=(
            jax.ShapeDtypeStruct((N, C), jnp.float32),
            jax.ShapeDtypeStruct((N, H, W, C), jnp.float32),
            jax.ShapeDtypeStruct((N, H, W, C), jnp.float32),
        ),
        scratch_shapes=[pltpu.VMEM((N, C), jnp.float32)],
        compiler_params=pltpu.CompilerParams(
            vmem_limit_bytes=60 * 1024 * 1024),
    )(mask, W1, b1.reshape(1, C), W2, b2.reshape(1, C),
      Wfc, bfc.reshape(1, C), lst_t, gui_t)

    return (out_lst_t.transpose(0, 3, 1, 2),
            out_gui_t.transpose(0, 3, 1, 2), m)
